# baseline (speedup 1.0000x reference)
import functools

import jax
import jax.numpy as jnp
from jax import lax
from jax.experimental import pallas as pl
from jax.experimental.pallas import tpu as pltpu
from jax.experimental.pallas import tpu_sc as plsc

N_CORES = 2
N_SUBCORES = 16
N_WORKERS = N_CORES * N_SUBCORES
CHUNK = 104
NBUF = 4
LANES = 16


def _sc_knn_message(x, src_idx, dst_idx):
    E = src_idx.shape[0]
    D = x.shape[1]
    per_w = E // N_WORKERS
    n_full = per_w // CHUNK
    n_rounds = n_full // NBUF
    tail = per_w - n_full * CHUNK
    assert per_w * N_WORKERS == E and n_rounds * NBUF == n_full
    assert tail % 8 == 0

    mesh = plsc.VectorSubcoreMesh(
        core_axis_name="c", subcore_axis_name="s", num_cores=N_CORES
    )

    scratch = [
        pltpu.VMEM((per_w,), jnp.int32),
        pltpu.VMEM((per_w,), jnp.int32),
        pltpu.VMEM((NBUF, CHUNK, D), jnp.float32),
        pltpu.VMEM((NBUF, CHUNK, D), jnp.float32),
    ]
    scratch += [pltpu.SemaphoreType.DMA] * (2 * NBUF)

    @functools.partial(
        pl.kernel,
        mesh=mesh,
        out_type=jax.ShapeDtypeStruct((E, 2 * D), jnp.float32),
        scratch_types=scratch,
    )
    def k(x_hbm, sidx_hbm, didx_hbm, out_hbm, sidx_v, didx_v, srows_v, drows_v,
          *sems):
        gsem = sems[:NBUF]
        osem = sems[NBUF:]
        wid = lax.axis_index("s") * N_CORES + lax.axis_index("c")
        base_w = wid * per_w

        pltpu.sync_copy(sidx_hbm.at[pl.ds(base_w, per_w)], sidx_v)
        pltpu.sync_copy(didx_hbm.at[pl.ds(base_w, per_w)], didx_v)

        def fire_gather(g, b, n=CHUNK):
            off = (g * NBUF + b) * CHUNK
            pltpu.async_copy(x_hbm.at[sidx_v.at[pl.ds(off, n)]],
                             srows_v.at[b, pl.ds(0, n)], gsem[b])
            pltpu.async_copy(x_hbm.at[didx_v.at[pl.ds(off, n)]],
                             drows_v.at[b, pl.ds(0, n)], gsem[b])

        def wait_gather(b, n=CHUNK):
            dummy = x_hbm.at[pl.ds(0, n)]
            pltpu.make_async_copy(dummy, srows_v.at[b, pl.ds(0, n)],
                                  gsem[b]).wait()
            pltpu.make_async_copy(dummy, drows_v.at[b, pl.ds(0, n)],
                                  gsem[b]).wait()

        def fire_out(g, b, n=CHUNK):
            base = base_w + (g * NBUF + b) * CHUNK
            pltpu.async_copy(drows_v.at[b, pl.ds(0, n)],
                             out_hbm.at[pl.ds(base, n), pl.ds(0, D)], osem[b])
            pltpu.async_copy(srows_v.at[b, pl.ds(0, n)],
                             out_hbm.at[pl.ds(base, n), pl.ds(D, D)], osem[b])

        def wait_out(b, n=CHUNK):
            dummy = out_hbm.at[pl.ds(0, n), pl.ds(0, D)]
            pltpu.make_async_copy(srows_v.at[b, pl.ds(0, n)], dummy,
                                  osem[b]).wait()
            pltpu.make_async_copy(drows_v.at[b, pl.ds(0, n)], dummy,
                                  osem[b]).wait()

        def compute(b, n=CHUNK):
            def edge_body(e, carry):
                for grp in range(D // LANES):
                    sl = pl.ds(grp * LANES, LANES)
                    s = srows_v[b, e, sl]
                    d = drows_v[b, e, sl]
                    drows_v[b, e, sl] = s - d
                return carry

            lax.fori_loop(0, n, edge_body, 0, unroll=2)

        for b in range(NBUF):
            fire_gather(0, b)

        def round_body(g, carry):
            for b in range(NBUF):
                wait_gather(b)
                compute(b)
                fire_out(g, b)
            for b in range(NBUF):
                wait_out(b)

                @pl.when(g + 1 < n_rounds)
                def _():
                    fire_gather(g + 1, b)

            return carry

        lax.fori_loop(0, n_rounds, round_body, 0)

        fire_gather(n_rounds, 0, tail)
        wait_gather(0, tail)
        compute(0, tail)
        fire_out(n_rounds, 0, tail)
        wait_out(0, tail)

    return k(x, src_idx, dst_idx)


def kernel(x, edge_index):
    src = edge_index[0].astype(jnp.int32)
    dst = edge_index[1].astype(jnp.int32)
    return _sc_knn_message(x, src, dst)


# speedup vs baseline: 1.1499x; 1.1499x over previous
import functools

import jax
import jax.numpy as jnp
from jax import lax
from jax.experimental import pallas as pl
from jax.experimental.pallas import tpu as pltpu
from jax.experimental.pallas import tpu_sc as plsc

N_CORES = 2
N_SUBCORES = 16
N_WORKERS = N_CORES * N_SUBCORES
CHUNK = 128
NBUF = 3
LANES = 16


def _sc_knn_message(x, src_idx, dst_idx):
    E = src_idx.shape[0]
    D = x.shape[1]
    per_w = E // N_WORKERS
    n_full = per_w // CHUNK
    n_rounds = n_full // NBUF
    tail = per_w - n_full * CHUNK
    assert per_w * N_WORKERS == E and n_rounds * NBUF == n_full
    assert tail % 8 == 0

    mesh = plsc.VectorSubcoreMesh(
        core_axis_name="c", subcore_axis_name="s", num_cores=N_CORES
    )

    scratch = [
        pltpu.VMEM((per_w,), jnp.int32),
        pltpu.VMEM((per_w,), jnp.int32),
        pltpu.VMEM((NBUF, CHUNK, D), jnp.float32),
        pltpu.VMEM((NBUF, CHUNK, D), jnp.float32),
    ]
    scratch += [pltpu.SemaphoreType.DMA] * (2 * NBUF)

    @functools.partial(
        pl.kernel,
        mesh=mesh,
        out_type=jax.ShapeDtypeStruct((E, 2 * D), jnp.float32),
        scratch_types=scratch,
    )
    def k(x_hbm, sidx_hbm, didx_hbm, out_hbm, sidx_v, didx_v, srows_v, drows_v,
          *sems):
        gsem = sems[:NBUF]
        osem = sems[NBUF:]
        wid = lax.axis_index("s") * N_CORES + lax.axis_index("c")
        base_w = wid * per_w

        pltpu.sync_copy(sidx_hbm.at[pl.ds(base_w, per_w)], sidx_v)
        pltpu.sync_copy(didx_hbm.at[pl.ds(base_w, per_w)], didx_v)

        def fire_gather(g, b, n=CHUNK):
            off = (g * NBUF + b) * CHUNK
            pltpu.async_copy(x_hbm.at[sidx_v.at[pl.ds(off, n)]],
                             srows_v.at[b, pl.ds(0, n)], gsem[b])
            pltpu.async_copy(x_hbm.at[didx_v.at[pl.ds(off, n)]],
                             drows_v.at[b, pl.ds(0, n)], gsem[b])

        def wait_gather(b, n=CHUNK):
            dummy = x_hbm.at[pl.ds(0, n)]
            pltpu.make_async_copy(dummy, srows_v.at[b, pl.ds(0, n)],
                                  gsem[b]).wait()
            pltpu.make_async_copy(dummy, drows_v.at[b, pl.ds(0, n)],
                                  gsem[b]).wait()

        def fire_out(g, b, n=CHUNK):
            base = base_w + (g * NBUF + b) * CHUNK
            pltpu.async_copy(drows_v.at[b, pl.ds(0, n)],
                             out_hbm.at[pl.ds(base, n), pl.ds(0, D)], osem[b])
            pltpu.async_copy(srows_v.at[b, pl.ds(0, n)],
                             out_hbm.at[pl.ds(base, n), pl.ds(D, D)], osem[b])

        def wait_out(b, n=CHUNK):
            dummy = out_hbm.at[pl.ds(0, n), pl.ds(0, D)]
            pltpu.make_async_copy(srows_v.at[b, pl.ds(0, n)], dummy,
                                  osem[b]).wait()
            pltpu.make_async_copy(drows_v.at[b, pl.ds(0, n)], dummy,
                                  osem[b]).wait()

        def compute(b, n=CHUNK):
            def edge_body(e, carry):
                for grp in range(D // LANES):
                    sl = pl.ds(grp * LANES, LANES)
                    s = srows_v[b, e, sl]
                    d = drows_v[b, e, sl]
                    drows_v[b, e, sl] = s - d
                return carry

            lax.fori_loop(0, n, edge_body, 0, unroll=2)

        for b in range(NBUF):
            fire_gather(0, b)

        def round_body(g, carry):
            for b in range(NBUF):
                wait_gather(b)
                compute(b)
                fire_out(g, b)
            for b in range(NBUF):
                wait_out(b)

                @pl.when(g + 1 < n_rounds)
                def _():
                    fire_gather(g + 1, b)

            return carry

        lax.fori_loop(0, n_rounds, round_body, 0)

        fire_gather(n_rounds, 0, tail)
        wait_gather(0, tail)
        compute(0, tail)
        fire_out(n_rounds, 0, tail)
        wait_out(0, tail)

    return k(x, src_idx, dst_idx)


def kernel(x, edge_index):
    src = edge_index[0].astype(jnp.int32)
    dst = edge_index[1].astype(jnp.int32)
    return _sc_knn_message(x, src, dst)
